# baseline (speedup 1.0000x reference)
import jax
import jax.numpy as jnp
from jax import lax
from jax.experimental import pallas as pl
from jax.experimental.pallas import tpu as pltpu
from jax.experimental.pallas import tpu_sc as plsc

_MAX_DET = 300
_OUTW = 304
_SCORE_THRESH = 0.3
_L = 16
_GS = 8
_SHARD = 640
_NSL = _SHARD // _L
_BIG = 1 << 30


def _sc_body(scal_hbm, x1_hbm, y1_hbm, x2_hbm, y2_hbm, sc_hbm, lb_hbm,
             out_hbm,
             scal_v, ox1_v, oy1_v, ox2_v, oy2_v,
             cx1_v, cy1_v, cx2_v, cy2_v, aj_v, alive_v, sc_v, lb_v,
             rec_v, rb_v, o_v, pub_sh):
    c = lax.axis_index("c")
    s = lax.axis_index("s")
    grp = s // _GS
    g = s % _GS
    img = c * 2 + grp
    base = g * _SHARD
    lane = lax.iota(jnp.int32, _L)
    neg_inf = jnp.float32(-jnp.inf)
    pos_inf = jnp.float32(jnp.inf)
    ninf_vec = jnp.full((_L,), neg_inf, jnp.float32)
    pinf_vec = jnp.full((_L,), pos_inf, jnp.float32)
    zero_i = jnp.zeros((_L,), jnp.int32)

    pltpu.sync_copy(scal_hbm, scal_v)
    sv = scal_v[...]
    imgf = sv[0]
    thr = sv[1]
    cw0 = sv[2]
    cw1 = sv[3]
    pltpu.sync_copy(x1_hbm.at[img, pl.ds(base, _SHARD)], cx1_v)
    pltpu.sync_copy(y1_hbm.at[img, pl.ds(base, _SHARD)], cy1_v)
    pltpu.sync_copy(x2_hbm.at[img, pl.ds(base, _SHARD)], cx2_v)
    pltpu.sync_copy(y2_hbm.at[img, pl.ds(base, _SHARD)], cy2_v)
    pltpu.sync_copy(sc_hbm.at[img, pl.ds(base, _SHARD)], sc_v)
    pltpu.sync_copy(lb_hbm.at[img, pl.ds(base, _SHARD)], lb_v)

    def prep(si, c2):
        mv1, mi1, mv2, mi2 = c2
        d = pl.ds(si * _L, _L)
        lb = lb_v[d]
        lbf = lb.astype(jnp.float32)
        off = lbf * (imgf + 1.0)
        cx1 = jnp.minimum(jnp.maximum(cx1_v[d], 0.0), imgf)
        cy1 = jnp.minimum(jnp.maximum(cy1_v[d], 0.0), imgf)
        cx2 = jnp.minimum(jnp.maximum(cx2_v[d], 0.0), imgf)
        cy2 = jnp.minimum(jnp.maximum(cy2_v[d], 0.0), imgf)
        cx1_v[d] = cx1
        cy1_v[d] = cy1
        cx2_v[d] = cx2
        cy2_v[d] = cy2
        x1 = cx1 + off
        y1 = cy1 + off
        x2 = cx2 + off
        y2 = cy2 + off
        ox1_v[d] = x1
        oy1_v[d] = y1
        ox2_v[d] = x2
        oy2_v[d] = y2
        aj_v[d] = (x2 - x1) * (y2 - y1)
        sc = sc_v[d]
        w = sc * jnp.where(lb == 0, cw0, cw1)
        a = jnp.where(sc > _SCORE_THRESH, w, neg_inf)
        alive_v[d] = a
        ii = lane + si * _L
        b1 = a > mv1
        dv = jnp.where(b1, mv1, a)
        di = jnp.where(b1, mi1, ii)
        b2 = dv > mv2
        return (jnp.where(b1, a, mv1), jnp.where(b1, ii, mi1),
                jnp.where(b2, dv, mv2), jnp.where(b2, di, mi2))

    mv10, mi10, mv20, mi20 = lax.fori_loop(
        0, _NSL, prep, (ninf_vec, zero_i, ninf_vec, zero_i))

    @pl.when(g == 0)
    def _():
        z = jnp.where(lane == 7, jnp.float32(-1.0), jnp.float32(0.0))

        def zinit(si, _):
            o_v[si] = z
            return 0

        lax.fori_loop(0, _OUTW, zinit, 0)

    rec_v[0] = ninf_vec
    rec_v[1] = ninf_vec
    pltpu.sync_copy(rec_v, pub_sh.at[1, grp, pl.ds(2 * g, 2)])
    pltpu.sync_copy(rec_v, pub_sh.at[2, grp, pl.ds(2 * g, 2)])
    plsc.subcore_barrier()

    def build_rec(m, win):
        winv = jnp.full((_L,), win, jnp.int32)
        rec = jnp.full((_L,), m, jnp.float32)
        rec = jnp.where(lane == 1, (base + win).astype(jnp.float32), rec)
        rec = jnp.where(lane == 2, plsc.load_gather(ox1_v, [winv]), rec)
        rec = jnp.where(lane == 3, plsc.load_gather(oy1_v, [winv]), rec)
        rec = jnp.where(lane == 4, plsc.load_gather(ox2_v, [winv]), rec)
        rec = jnp.where(lane == 5, plsc.load_gather(oy2_v, [winv]), rec)
        rec = jnp.where(lane == 6, plsc.load_gather(sc_v, [winv]), rec)
        rec = jnp.where(lane == 7,
                        plsc.load_gather(lb_v, [winv]).astype(jnp.float32), rec)
        rec = jnp.where(lane == 8, plsc.load_gather(cx1_v, [winv]), rec)
        rec = jnp.where(lane == 9, plsc.load_gather(cy1_v, [winv]), rec)
        rec = jnp.where(lane == 10, plsc.load_gather(cx2_v, [winv]), rec)
        rec = jnp.where(lane == 11, plsc.load_gather(cy2_v, [winv]), rec)
        return rec

    def step(t, carry):
        count, mv1, mi1, mv2, mi2 = carry
        m1 = jnp.max(mv1)
        i1 = jnp.min(jnp.where(mv1 == m1, mi1, _BIG))
        mv1x = jnp.where((mv1 == m1) & (mi1 == i1), neg_inf, mv1)
        bb = (mv2 > mv1x) | ((mv2 == mv1x) & (mi2 < mi1))
        vc = jnp.where(bb, mv2, mv1x)
        ic = jnp.where(bb, mi2, mi1)
        m2 = jnp.max(vc)
        i2 = jnp.min(jnp.where(vc == m2, ic, _BIG))
        rec_v[0] = build_rec(m1, i1)
        rec_v[1] = build_rec(m2, i2)

        p = t % 2 + 1
        pltpu.sync_copy(rec_v, pub_sh.at[p, grp, pl.ds(2 * g, 2)])
        plsc.subcore_barrier()
        pltpu.sync_copy(pub_sh.at[p, grp], rb_v)

        val16 = plsc.load_gather(rb_v, [lane, zero_i])
        gid16 = plsc.load_gather(rb_v, [lane, zero_i + 1])
        mW = jnp.max(val16)
        active = mW > neg_inf
        gW = jnp.min(jnp.where(val16 == mW, gid16, pos_inf))
        kW = jnp.min(jnp.where((val16 == mW) & (gid16 == gW), lane, _L))
        kWv = jnp.full((_L,), kW, jnp.int32)
        val16x = jnp.where(lane == kW, neg_inf, val16)
        mS = jnp.max(val16x)
        gS = jnp.min(jnp.where(val16x == mS, gid16, pos_inf))
        kS = jnp.min(jnp.where((val16x == mS) & (gid16 == gS), lane, _L))
        kSv = jnp.full((_L,), kS, jnp.int32)

        bx1 = jnp.where(active, plsc.load_gather(rb_v, [kWv, zero_i + 2]), pinf_vec)
        by1 = jnp.where(active, plsc.load_gather(rb_v, [kWv, zero_i + 3]), pinf_vec)
        bx2 = jnp.where(active, plsc.load_gather(rb_v, [kWv, zero_i + 4]), ninf_vec)
        by2 = jnp.where(active, plsc.load_gather(rb_v, [kWv, zero_i + 5]), ninf_vec)
        a_iv = (bx2 - bx1) * (by2 - by1)
        wg1 = jnp.where(active, gW.astype(jnp.int32), -1)

        sx1 = plsc.load_gather(rb_v, [kSv, zero_i + 2])
        sy1 = plsc.load_gather(rb_v, [kSv, zero_i + 3])
        sx2 = plsc.load_gather(rb_v, [kSv, zero_i + 4])
        sy2 = plsc.load_gather(rb_v, [kSv, zero_i + 5])
        a_sv = (sx2 - sx1) * (sy2 - sy1)
        qx1 = jnp.maximum(bx1, sx1)
        qy1 = jnp.maximum(by1, sy1)
        qx2 = jnp.minimum(bx2, sx2)
        qy2 = jnp.minimum(by2, sy2)
        qint = (jnp.maximum(qx2 - qx1, 0.0) * jnp.maximum(qy2 - qy1, 0.0))
        qiou = qint / (a_iv + a_sv - qint + 1e-9)
        emit2 = active & (mS > neg_inf) & jnp.logical_not(qiou[0] > thr)
        cx1s = jnp.where(emit2, sx1, pinf_vec)
        cy1s = jnp.where(emit2, sy1, pinf_vec)
        cx2s = jnp.where(emit2, sx2, ninf_vec)
        cy2s = jnp.where(emit2, sy2, ninf_vec)
        a_sv2 = (cx2s - cx1s) * (cy2s - cy1s)
        wg2 = jnp.where(emit2, gS.astype(jnp.int32), -1)

        def fused(si, c2):
            nv1, ni1, nv2, ni2 = c2
            d = pl.ds(si * _L, _L)
            a = alive_v[d]
            x1 = ox1_v[d]
            y1 = oy1_v[d]
            x2 = ox2_v[d]
            y2 = oy2_v[d]
            aj = aj_v[d]
            xx1 = jnp.maximum(bx1, x1)
            yy1 = jnp.maximum(by1, y1)
            xx2 = jnp.minimum(bx2, x2)
            yy2 = jnp.minimum(by2, y2)
            inter = (jnp.maximum(xx2 - xx1, 0.0)
                     * jnp.maximum(yy2 - yy1, 0.0))
            iou = inter / (a_iv + aj - inter + 1e-9)
            ux1 = jnp.maximum(cx1s, x1)
            uy1 = jnp.maximum(cy1s, y1)
            ux2 = jnp.minimum(cx2s, x2)
            uy2 = jnp.minimum(cy2s, y2)
            uint = (jnp.maximum(ux2 - ux1, 0.0)
                    * jnp.maximum(uy2 - uy1, 0.0))
            uiou = uint / (a_sv2 + aj - uint + 1e-9)
            ii = lane + si * _L
            gidx = ii + base
            kill = ((iou > thr) | (uiou > thr)
                    | (gidx == wg1) | (gidx == wg2))
            a = jnp.where(kill, neg_inf, a)
            alive_v[d] = a
            b1 = a > nv1
            dv = jnp.where(b1, nv1, a)
            di = jnp.where(b1, ni1, ii)
            b2 = dv > nv2
            return (jnp.where(b1, a, nv1), jnp.where(b1, ii, ni1),
                    jnp.where(b2, dv, nv2), jnp.where(b2, di, ni2))

        nmv1, nmi1, nmv2, nmi2 = plsc.parallel_loop(
            0, _NSL, unroll=4,
            carry=(ninf_vec, zero_i, ninf_vec, zero_i))(fused)

        @pl.when(active & (g == 0) & (count < _MAX_DET))
        def _():
            o_v[count] = plsc.load_gather(rb_v, [kWv, lane])

        count1 = count + jnp.where(active, 1, 0)

        @pl.when(emit2 & (g == 0) & (count1 < _MAX_DET))
        def _():
            o_v[count1] = plsc.load_gather(rb_v, [kSv, lane])

        count2 = count1 + jnp.where(emit2, 1, 0)
        return (count2, nmv1, nmi1, nmv2, nmi2)

    lax.fori_loop(0, _MAX_DET + 2, step,
                  (jnp.int32(0), mv10, mi10, mv20, mi20))

    @pl.when(g == 0)
    def _():
        pltpu.sync_copy(o_v, out_hbm.at[img])


def kernel(boxes, scores, labels, img_size, nms_thresh, class_weights):
    B, N, _ = boxes.shape
    Np = _GS * _SHARD
    pad = Np - N

    thr = jax.nn.sigmoid(jnp.asarray(nms_thresh, jnp.float32))
    imgf = jnp.asarray(img_size, jnp.float32)
    cw = jnp.asarray(class_weights, jnp.float32)
    scal = jnp.zeros((16,), jnp.float32)
    scal = scal.at[0].set(imgf).at[1].set(thr)
    scal = scal.at[2].set(cw[0]).at[3].set(cw[1])

    x1 = jnp.pad(boxes[:, :, 0], ((0, 0), (0, pad)))
    y1 = jnp.pad(boxes[:, :, 1], ((0, 0), (0, pad)))
    x2 = jnp.pad(boxes[:, :, 2], ((0, 0), (0, pad)))
    y2 = jnp.pad(boxes[:, :, 3], ((0, 0), (0, pad)))
    scp = jnp.pad(scores, ((0, 0), (0, pad)), constant_values=-1.0)
    lbp = jnp.pad(labels.astype(jnp.int32), ((0, 0), (0, pad)))

    mesh = plsc.VectorSubcoreMesh(core_axis_name="c", subcore_axis_name="s",
                                  num_cores=2, num_subcores=16)
    run = pl.kernel(
        _sc_body,
        out_type=jax.ShapeDtypeStruct((B, _OUTW, _L), jnp.float32),
        mesh=mesh,
        compiler_params=pltpu.CompilerParams(needs_layout_passes=False),
        scratch_types=[
            pltpu.VMEM((_L,), jnp.float32),
            pltpu.VMEM((_SHARD,), jnp.float32),
            pltpu.VMEM((_SHARD,), jnp.float32),
            pltpu.VMEM((_SHARD,), jnp.float32),
            pltpu.VMEM((_SHARD,), jnp.float32),
            pltpu.VMEM((_SHARD,), jnp.float32),
            pltpu.VMEM((_SHARD,), jnp.float32),
            pltpu.VMEM((_SHARD,), jnp.float32),
            pltpu.VMEM((_SHARD,), jnp.float32),
            pltpu.VMEM((_SHARD,), jnp.float32),
            pltpu.VMEM((_SHARD,), jnp.float32),
            pltpu.VMEM((_SHARD,), jnp.float32),
            pltpu.VMEM((_SHARD,), jnp.int32),
            pltpu.VMEM((2, _L), jnp.float32),
            pltpu.VMEM((2 * _GS, _L), jnp.float32),
            pltpu.VMEM((_OUTW, _L), jnp.float32),
            pltpu.VMEM_SHARED((3, 2, 2 * _GS, _L), jnp.float32),
        ],
    )
    out = run(scal, x1, y1, x2, y2, scp, lbp)

    det = out[:, :_MAX_DET, :]
    out_boxes = det[:, :, 8:12]
    out_scores = det[:, :, 6]
    out_labels = det[:, :, 7].astype(jnp.int32)
    return out_boxes, out_scores, out_labels


# speedup vs baseline: 1.6567x; 1.6567x over previous
import jax
import jax.numpy as jnp
from jax import lax
from jax.experimental import pallas as pl
from jax.experimental.pallas import tpu as pltpu
from jax.experimental.pallas import tpu_sc as plsc

_MAX_DET = 300
_OUTW = 304
_SCORE_THRESH = 0.3
_L = 16
_GS = 8
_SHARD = 640
_NSL = _SHARD // _L
_BIG = 1 << 30


def _sc_body(scal_hbm, x1_hbm, y1_hbm, x2_hbm, y2_hbm, sc_hbm, lb_hbm,
             out_hbm,
             scal_v, ox1_v, oy1_v, ox2_v, oy2_v,
             cx1_v, cy1_v, cx2_v, cy2_v, aj_v, alive_v, sc_v, lb_v,
             rec_v, rb_v, o_v, pub_sh):
    c = lax.axis_index("c")
    s = lax.axis_index("s")
    grp = s // _GS
    g = s % _GS
    img = c * 2 + grp
    base = g * _SHARD
    lane = lax.iota(jnp.int32, _L)
    neg_inf = jnp.float32(-jnp.inf)
    pos_inf = jnp.float32(jnp.inf)
    ninf_vec = jnp.full((_L,), neg_inf, jnp.float32)
    pinf_vec = jnp.full((_L,), pos_inf, jnp.float32)
    zero_i = jnp.zeros((_L,), jnp.int32)

    pltpu.sync_copy(scal_hbm, scal_v)
    sv = scal_v[...]
    imgf = sv[0]
    thr = sv[1]
    cw0 = sv[2]
    cw1 = sv[3]
    pltpu.sync_copy(x1_hbm.at[img, pl.ds(base, _SHARD)], cx1_v)
    pltpu.sync_copy(y1_hbm.at[img, pl.ds(base, _SHARD)], cy1_v)
    pltpu.sync_copy(x2_hbm.at[img, pl.ds(base, _SHARD)], cx2_v)
    pltpu.sync_copy(y2_hbm.at[img, pl.ds(base, _SHARD)], cy2_v)
    pltpu.sync_copy(sc_hbm.at[img, pl.ds(base, _SHARD)], sc_v)
    pltpu.sync_copy(lb_hbm.at[img, pl.ds(base, _SHARD)], lb_v)

    def prep(si, c2):
        mv1, mi1, mv2, mi2 = c2
        d = pl.ds(si * _L, _L)
        lb = lb_v[d]
        lbf = lb.astype(jnp.float32)
        off = lbf * (imgf + 1.0)
        cx1 = jnp.minimum(jnp.maximum(cx1_v[d], 0.0), imgf)
        cy1 = jnp.minimum(jnp.maximum(cy1_v[d], 0.0), imgf)
        cx2 = jnp.minimum(jnp.maximum(cx2_v[d], 0.0), imgf)
        cy2 = jnp.minimum(jnp.maximum(cy2_v[d], 0.0), imgf)
        cx1_v[d] = cx1
        cy1_v[d] = cy1
        cx2_v[d] = cx2
        cy2_v[d] = cy2
        x1 = cx1 + off
        y1 = cy1 + off
        x2 = cx2 + off
        y2 = cy2 + off
        ox1_v[d] = x1
        oy1_v[d] = y1
        ox2_v[d] = x2
        oy2_v[d] = y2
        aj_v[d] = (x2 - x1) * (y2 - y1)
        sc = sc_v[d]
        w = sc * jnp.where(lb == 0, cw0, cw1)
        a = jnp.where(sc > _SCORE_THRESH, w, neg_inf)
        alive_v[d] = a
        ii = lane + si * _L
        b1 = a > mv1
        dv = jnp.where(b1, mv1, a)
        di = jnp.where(b1, mi1, ii)
        b2 = dv > mv2
        return (jnp.where(b1, a, mv1), jnp.where(b1, ii, mi1),
                jnp.where(b2, dv, mv2), jnp.where(b2, di, mi2))

    mv10, mi10, mv20, mi20 = lax.fori_loop(
        0, _NSL, prep, (ninf_vec, zero_i, ninf_vec, zero_i))

    @pl.when(g == 0)
    def _():
        z = jnp.where(lane == 7, jnp.float32(-1.0), jnp.float32(0.0))

        def zinit(si, _):
            o_v[si] = z
            return 0

        lax.fori_loop(0, _OUTW, zinit, 0)

    rec_v[0] = ninf_vec
    rec_v[1] = ninf_vec
    pltpu.sync_copy(rec_v, pub_sh.at[1, grp, pl.ds(2 * g, 2)])
    pltpu.sync_copy(rec_v, pub_sh.at[2, grp, pl.ds(2 * g, 2)])
    plsc.subcore_barrier()

    def build_rec(m, win):
        winv = jnp.full((_L,), win, jnp.int32)
        rec = jnp.full((_L,), m, jnp.float32)
        rec = jnp.where(lane == 1, (base + win).astype(jnp.float32), rec)
        rec = jnp.where(lane == 2, plsc.load_gather(ox1_v, [winv]), rec)
        rec = jnp.where(lane == 3, plsc.load_gather(oy1_v, [winv]), rec)
        rec = jnp.where(lane == 4, plsc.load_gather(ox2_v, [winv]), rec)
        rec = jnp.where(lane == 5, plsc.load_gather(oy2_v, [winv]), rec)
        rec = jnp.where(lane == 6, plsc.load_gather(sc_v, [winv]), rec)
        rec = jnp.where(lane == 7,
                        plsc.load_gather(lb_v, [winv]).astype(jnp.float32), rec)
        rec = jnp.where(lane == 8, plsc.load_gather(cx1_v, [winv]), rec)
        rec = jnp.where(lane == 9, plsc.load_gather(cy1_v, [winv]), rec)
        rec = jnp.where(lane == 10, plsc.load_gather(cx2_v, [winv]), rec)
        rec = jnp.where(lane == 11, plsc.load_gather(cy2_v, [winv]), rec)
        return rec

    def step_cond(carry):
        t, go, count, done, mv1, mi1, mv2, mi2 = carry
        return go & (t < _MAX_DET + 2)

    def step(carry):
        t, go, count, done, mv1, mi1, mv2, mi2 = carry
        m1 = jnp.max(mv1)
        i1 = jnp.min(jnp.where(mv1 == m1, mi1, _BIG))
        mv1x = jnp.where((mv1 == m1) & (mi1 == i1), neg_inf, mv1)
        bb = (mv2 > mv1x) | ((mv2 == mv1x) & (mi2 < mi1))
        vc = jnp.where(bb, mv2, mv1x)
        ic = jnp.where(bb, mi2, mi1)
        m2 = jnp.max(vc)
        i2 = jnp.min(jnp.where(vc == m2, ic, _BIG))
        donef = jnp.where(done, jnp.float32(1.0), jnp.float32(0.0))
        rec_v[0] = jnp.where(lane == 12, donef, build_rec(m1, i1))
        rec_v[1] = jnp.where(lane == 12, donef, build_rec(m2, i2))

        p = t % 2 + 1
        pltpu.sync_copy(rec_v, pub_sh.at[p, grp, pl.ds(2 * g, 2)])
        plsc.subcore_barrier()
        pltpu.sync_copy(pub_sh.at[p], rb_v)

        grpv = jnp.full((_L,), grp, jnp.int32)
        ogrpv = jnp.full((_L,), 1 - grp, jnp.int32)
        val16 = plsc.load_gather(rb_v, [grpv, lane, zero_i])
        gid16 = plsc.load_gather(rb_v, [grpv, lane, zero_i + 1])
        other_done = plsc.load_gather(rb_v, [ogrpv, zero_i, zero_i + 12])[0] > 0.5
        go2 = jnp.logical_not(done & other_done)
        mW = jnp.max(val16)
        active = mW > neg_inf
        gW = jnp.min(jnp.where(val16 == mW, gid16, pos_inf))
        kW = jnp.min(jnp.where((val16 == mW) & (gid16 == gW), lane, _L))
        kWv = jnp.full((_L,), kW, jnp.int32)
        val16x = jnp.where(lane == kW, neg_inf, val16)
        mS = jnp.max(val16x)
        gS = jnp.min(jnp.where(val16x == mS, gid16, pos_inf))
        kS = jnp.min(jnp.where((val16x == mS) & (gid16 == gS), lane, _L))
        kSv = jnp.full((_L,), kS, jnp.int32)

        bx1 = jnp.where(active, plsc.load_gather(rb_v, [grpv, kWv, zero_i + 2]), pinf_vec)
        by1 = jnp.where(active, plsc.load_gather(rb_v, [grpv, kWv, zero_i + 3]), pinf_vec)
        bx2 = jnp.where(active, plsc.load_gather(rb_v, [grpv, kWv, zero_i + 4]), ninf_vec)
        by2 = jnp.where(active, plsc.load_gather(rb_v, [grpv, kWv, zero_i + 5]), ninf_vec)
        a_iv = (bx2 - bx1) * (by2 - by1)
        wg1 = jnp.where(active, gW.astype(jnp.int32), -1)

        sx1 = plsc.load_gather(rb_v, [grpv, kSv, zero_i + 2])
        sy1 = plsc.load_gather(rb_v, [grpv, kSv, zero_i + 3])
        sx2 = plsc.load_gather(rb_v, [grpv, kSv, zero_i + 4])
        sy2 = plsc.load_gather(rb_v, [grpv, kSv, zero_i + 5])
        a_sv = (sx2 - sx1) * (sy2 - sy1)
        qx1 = jnp.maximum(bx1, sx1)
        qy1 = jnp.maximum(by1, sy1)
        qx2 = jnp.minimum(bx2, sx2)
        qy2 = jnp.minimum(by2, sy2)
        qint = (jnp.maximum(qx2 - qx1, 0.0) * jnp.maximum(qy2 - qy1, 0.0))
        qiou = qint / (a_iv + a_sv - qint + 1e-9)
        emit2 = active & (mS > neg_inf) & jnp.logical_not(qiou[0] > thr)
        cx1s = jnp.where(emit2, sx1, pinf_vec)
        cy1s = jnp.where(emit2, sy1, pinf_vec)
        cx2s = jnp.where(emit2, sx2, ninf_vec)
        cy2s = jnp.where(emit2, sy2, ninf_vec)
        a_sv2 = (cx2s - cx1s) * (cy2s - cy1s)
        wg2 = jnp.where(emit2, gS.astype(jnp.int32), -1)

        def fused(si, c2):
            nv1, ni1, nv2, ni2 = c2
            d = pl.ds(si * _L, _L)
            a = alive_v[d]
            x1 = ox1_v[d]
            y1 = oy1_v[d]
            x2 = ox2_v[d]
            y2 = oy2_v[d]
            aj = aj_v[d]
            xx1 = jnp.maximum(bx1, x1)
            yy1 = jnp.maximum(by1, y1)
            xx2 = jnp.minimum(bx2, x2)
            yy2 = jnp.minimum(by2, y2)
            inter = (jnp.maximum(xx2 - xx1, 0.0)
                     * jnp.maximum(yy2 - yy1, 0.0))
            iou = inter / (a_iv + aj - inter + 1e-9)
            ux1 = jnp.maximum(cx1s, x1)
            uy1 = jnp.maximum(cy1s, y1)
            ux2 = jnp.minimum(cx2s, x2)
            uy2 = jnp.minimum(cy2s, y2)
            uint = (jnp.maximum(ux2 - ux1, 0.0)
                    * jnp.maximum(uy2 - uy1, 0.0))
            uiou = uint / (a_sv2 + aj - uint + 1e-9)
            ii = lane + si * _L
            gidx = ii + base
            kill = ((iou > thr) | (uiou > thr)
                    | (gidx == wg1) | (gidx == wg2))
            a = jnp.where(kill, neg_inf, a)
            alive_v[d] = a
            b1 = a > nv1
            dv = jnp.where(b1, nv1, a)
            di = jnp.where(b1, ni1, ii)
            b2 = dv > nv2
            return (jnp.where(b1, a, nv1), jnp.where(b1, ii, ni1),
                    jnp.where(b2, dv, nv2), jnp.where(b2, di, ni2))

        nmv1, nmi1, nmv2, nmi2 = plsc.parallel_loop(
            0, _NSL, unroll=4,
            carry=(ninf_vec, zero_i, ninf_vec, zero_i))(fused)

        @pl.when(active & (g == 0) & (count < _MAX_DET))
        def _():
            o_v[count] = plsc.load_gather(rb_v, [grpv, kWv, lane])

        count1 = count + jnp.where(active, 1, 0)

        @pl.when(emit2 & (g == 0) & (count1 < _MAX_DET))
        def _():
            o_v[count1] = plsc.load_gather(rb_v, [grpv, kSv, lane])

        count2 = count1 + jnp.where(emit2, 1, 0)
        done2 = (count2 >= _MAX_DET) | jnp.logical_not(active)
        return (t + 1, go2, count2, done2, nmv1, nmi1, nmv2, nmi2)

    lax.while_loop(step_cond, step,
                   (jnp.int32(0), jnp.bool_(True), jnp.int32(0),
                    jnp.bool_(False), mv10, mi10, mv20, mi20))

    @pl.when(g == 0)
    def _():
        pltpu.sync_copy(o_v, out_hbm.at[img])


def kernel(boxes, scores, labels, img_size, nms_thresh, class_weights):
    B, N, _ = boxes.shape
    Np = _GS * _SHARD
    pad = Np - N

    thr = jax.nn.sigmoid(jnp.asarray(nms_thresh, jnp.float32))
    imgf = jnp.asarray(img_size, jnp.float32)
    cw = jnp.asarray(class_weights, jnp.float32)
    scal = jnp.zeros((16,), jnp.float32)
    scal = scal.at[0].set(imgf).at[1].set(thr)
    scal = scal.at[2].set(cw[0]).at[3].set(cw[1])

    x1 = jnp.pad(boxes[:, :, 0], ((0, 0), (0, pad)))
    y1 = jnp.pad(boxes[:, :, 1], ((0, 0), (0, pad)))
    x2 = jnp.pad(boxes[:, :, 2], ((0, 0), (0, pad)))
    y2 = jnp.pad(boxes[:, :, 3], ((0, 0), (0, pad)))
    scp = jnp.pad(scores, ((0, 0), (0, pad)), constant_values=-1.0)
    lbp = jnp.pad(labels.astype(jnp.int32), ((0, 0), (0, pad)))

    mesh = plsc.VectorSubcoreMesh(core_axis_name="c", subcore_axis_name="s",
                                  num_cores=2, num_subcores=16)
    run = pl.kernel(
        _sc_body,
        out_type=jax.ShapeDtypeStruct((B, _OUTW, _L), jnp.float32),
        mesh=mesh,
        compiler_params=pltpu.CompilerParams(needs_layout_passes=False),
        scratch_types=[
            pltpu.VMEM((_L,), jnp.float32),
            pltpu.VMEM((_SHARD,), jnp.float32),
            pltpu.VMEM((_SHARD,), jnp.float32),
            pltpu.VMEM((_SHARD,), jnp.float32),
            pltpu.VMEM((_SHARD,), jnp.float32),
            pltpu.VMEM((_SHARD,), jnp.float32),
            pltpu.VMEM((_SHARD,), jnp.float32),
            pltpu.VMEM((_SHARD,), jnp.float32),
            pltpu.VMEM((_SHARD,), jnp.float32),
            pltpu.VMEM((_SHARD,), jnp.float32),
            pltpu.VMEM((_SHARD,), jnp.float32),
            pltpu.VMEM((_SHARD,), jnp.float32),
            pltpu.VMEM((_SHARD,), jnp.int32),
            pltpu.VMEM((2, _L), jnp.float32),
            pltpu.VMEM((2, 2 * _GS, _L), jnp.float32),
            pltpu.VMEM((_OUTW, _L), jnp.float32),
            pltpu.VMEM_SHARED((3, 2, 2 * _GS, _L), jnp.float32),
        ],
    )
    out = run(scal, x1, y1, x2, y2, scp, lbp)

    det = out[:, :_MAX_DET, :]
    out_boxes = det[:, :, 8:12]
    out_scores = det[:, :, 6]
    out_labels = det[:, :, 7].astype(jnp.int32)
    return out_boxes, out_scores, out_labels


# speedup vs baseline: 1.6828x; 1.0157x over previous
import jax
import jax.numpy as jnp
from jax import lax
from jax.experimental import pallas as pl
from jax.experimental.pallas import tpu as pltpu
from jax.experimental.pallas import tpu_sc as plsc

_MAX_DET = 300
_OUTW = 304
_SCORE_THRESH = 0.3
_L = 16
_GS = 8
_SHARD = 640
_NSL = _SHARD // _L
_BIG = 1 << 30


def _sc_body(scal_hbm, x1_hbm, y1_hbm, x2_hbm, y2_hbm, sc_hbm, lb_hbm,
             out_hbm,
             scal_v, ox1_v, oy1_v, ox2_v, oy2_v,
             cx1_v, cy1_v, cx2_v, cy2_v, aj_v, alive_v, sc_v, lb_v,
             rec_v, rb_v, o_v, pub_sh):
    c = lax.axis_index("c")
    s = lax.axis_index("s")
    grp = s // _GS
    g = s % _GS
    img = c * 2 + grp
    base = g * _SHARD
    lane = lax.iota(jnp.int32, _L)
    neg_inf = jnp.float32(-jnp.inf)
    pos_inf = jnp.float32(jnp.inf)
    ninf_vec = jnp.full((_L,), neg_inf, jnp.float32)
    pinf_vec = jnp.full((_L,), pos_inf, jnp.float32)
    zero_i = jnp.zeros((_L,), jnp.int32)

    pltpu.sync_copy(scal_hbm, scal_v)
    sv = scal_v[...]
    imgf = sv[0]
    thr = sv[1]
    cw0 = sv[2]
    cw1 = sv[3]
    pltpu.sync_copy(x1_hbm.at[img, pl.ds(base, _SHARD)], cx1_v)
    pltpu.sync_copy(y1_hbm.at[img, pl.ds(base, _SHARD)], cy1_v)
    pltpu.sync_copy(x2_hbm.at[img, pl.ds(base, _SHARD)], cx2_v)
    pltpu.sync_copy(y2_hbm.at[img, pl.ds(base, _SHARD)], cy2_v)
    pltpu.sync_copy(sc_hbm.at[img, pl.ds(base, _SHARD)], sc_v)
    pltpu.sync_copy(lb_hbm.at[img, pl.ds(base, _SHARD)], lb_v)

    def prep(si, c2):
        mv1, mi1, mv2, mi2 = c2
        d = pl.ds(si * _L, _L)
        lb = lb_v[d]
        lbf = lb.astype(jnp.float32)
        off = lbf * (imgf + 1.0)
        cx1 = jnp.minimum(jnp.maximum(cx1_v[d], 0.0), imgf)
        cy1 = jnp.minimum(jnp.maximum(cy1_v[d], 0.0), imgf)
        cx2 = jnp.minimum(jnp.maximum(cx2_v[d], 0.0), imgf)
        cy2 = jnp.minimum(jnp.maximum(cy2_v[d], 0.0), imgf)
        cx1_v[d] = cx1
        cy1_v[d] = cy1
        cx2_v[d] = cx2
        cy2_v[d] = cy2
        x1 = cx1 + off
        y1 = cy1 + off
        x2 = cx2 + off
        y2 = cy2 + off
        ox1_v[d] = x1
        oy1_v[d] = y1
        ox2_v[d] = x2
        oy2_v[d] = y2
        aj_v[d] = (x2 - x1) * (y2 - y1)
        sc = sc_v[d]
        w = sc * jnp.where(lb == 0, cw0, cw1)
        a = jnp.where(sc > _SCORE_THRESH, w, neg_inf)
        alive_v[d] = a
        ii = lane + si * _L
        b1 = a > mv1
        dv = jnp.where(b1, mv1, a)
        di = jnp.where(b1, mi1, ii)
        b2 = dv > mv2
        return (jnp.where(b1, a, mv1), jnp.where(b1, ii, mi1),
                jnp.where(b2, dv, mv2), jnp.where(b2, di, mi2))

    mv10, mi10, mv20, mi20 = lax.fori_loop(
        0, _NSL, prep, (ninf_vec, zero_i, ninf_vec, zero_i))

    @pl.when(g == 0)
    def _():
        z = jnp.where(lane == 7, jnp.float32(-1.0), jnp.float32(0.0))

        def zinit(si, _):
            o_v[si] = z
            return 0

        lax.fori_loop(0, _OUTW, zinit, 0)

    rec_v[0] = ninf_vec
    rec_v[1] = ninf_vec
    pltpu.sync_copy(rec_v, pub_sh.at[1, grp, pl.ds(2 * g, 2)])
    pltpu.sync_copy(rec_v, pub_sh.at[2, grp, pl.ds(2 * g, 2)])
    plsc.subcore_barrier()

    def build_rec(m, win):
        winv = jnp.full((_L,), win, jnp.int32)
        rec = jnp.full((_L,), m, jnp.float32)
        rec = jnp.where(lane == 1, (base + win).astype(jnp.float32), rec)
        rec = jnp.where(lane == 2, plsc.load_gather(ox1_v, [winv]), rec)
        rec = jnp.where(lane == 3, plsc.load_gather(oy1_v, [winv]), rec)
        rec = jnp.where(lane == 4, plsc.load_gather(ox2_v, [winv]), rec)
        rec = jnp.where(lane == 5, plsc.load_gather(oy2_v, [winv]), rec)
        rec = jnp.where(lane == 6, plsc.load_gather(sc_v, [winv]), rec)
        rec = jnp.where(lane == 7,
                        plsc.load_gather(lb_v, [winv]).astype(jnp.float32), rec)
        rec = jnp.where(lane == 8, plsc.load_gather(cx1_v, [winv]), rec)
        rec = jnp.where(lane == 9, plsc.load_gather(cy1_v, [winv]), rec)
        rec = jnp.where(lane == 10, plsc.load_gather(cx2_v, [winv]), rec)
        rec = jnp.where(lane == 11, plsc.load_gather(cy2_v, [winv]), rec)
        return rec

    def step_cond(carry):
        t, go, count, done, mv1, mi1, mv2, mi2 = carry
        return go & (t < _MAX_DET + 2)

    def step(carry):
        t, go, count, done, mv1, mi1, mv2, mi2 = carry
        m1 = jnp.max(mv1)
        i1 = jnp.min(jnp.where(mv1 == m1, mi1, _BIG))
        mv1x = jnp.where((mv1 == m1) & (mi1 == i1), neg_inf, mv1)
        bb = (mv2 > mv1x) | ((mv2 == mv1x) & (mi2 < mi1))
        vc = jnp.where(bb, mv2, mv1x)
        ic = jnp.where(bb, mi2, mi1)
        m2 = jnp.max(vc)
        i2 = jnp.min(jnp.where(vc == m2, ic, _BIG))
        donef = jnp.where(done, jnp.float32(1.0), jnp.float32(0.0))
        rec_v[0] = jnp.where(lane == 12, donef, build_rec(m1, i1))
        rec_v[1] = jnp.where(lane == 12, donef, build_rec(m2, i2))

        p = t % 2 + 1
        pltpu.sync_copy(rec_v, pub_sh.at[p, grp, pl.ds(2 * g, 2)])
        plsc.subcore_barrier()
        pltpu.sync_copy(pub_sh.at[p], rb_v)

        grpv = jnp.full((_L,), grp, jnp.int32)
        ogrpv = jnp.full((_L,), 1 - grp, jnp.int32)
        val16 = plsc.load_gather(rb_v, [grpv, lane, zero_i])
        gid16 = plsc.load_gather(rb_v, [grpv, lane, zero_i + 1])
        other_done = plsc.load_gather(rb_v, [ogrpv, zero_i, zero_i + 12])[0] > 0.5
        go2 = jnp.logical_not(done & other_done)
        mW = jnp.max(val16)
        active = mW > neg_inf
        gW = jnp.min(jnp.where(val16 == mW, gid16, pos_inf))
        kW = jnp.min(jnp.where((val16 == mW) & (gid16 == gW), lane, _L))
        kWv = jnp.full((_L,), kW, jnp.int32)
        val16x = jnp.where(lane == kW, neg_inf, val16)
        mS = jnp.max(val16x)
        gS = jnp.min(jnp.where(val16x == mS, gid16, pos_inf))
        kS = jnp.min(jnp.where((val16x == mS) & (gid16 == gS), lane, _L))
        kSv = jnp.full((_L,), kS, jnp.int32)

        bx1 = jnp.where(active, plsc.load_gather(rb_v, [grpv, kWv, zero_i + 2]), pinf_vec)
        by1 = jnp.where(active, plsc.load_gather(rb_v, [grpv, kWv, zero_i + 3]), pinf_vec)
        bx2 = jnp.where(active, plsc.load_gather(rb_v, [grpv, kWv, zero_i + 4]), ninf_vec)
        by2 = jnp.where(active, plsc.load_gather(rb_v, [grpv, kWv, zero_i + 5]), ninf_vec)
        a_iv = (bx2 - bx1) * (by2 - by1)
        wg1 = jnp.where(active, gW.astype(jnp.int32), -1)

        sx1 = plsc.load_gather(rb_v, [grpv, kSv, zero_i + 2])
        sy1 = plsc.load_gather(rb_v, [grpv, kSv, zero_i + 3])
        sx2 = plsc.load_gather(rb_v, [grpv, kSv, zero_i + 4])
        sy2 = plsc.load_gather(rb_v, [grpv, kSv, zero_i + 5])
        a_sv = (sx2 - sx1) * (sy2 - sy1)
        qx1 = jnp.maximum(bx1, sx1)
        qy1 = jnp.maximum(by1, sy1)
        qx2 = jnp.minimum(bx2, sx2)
        qy2 = jnp.minimum(by2, sy2)
        qint = (jnp.maximum(qx2 - qx1, 0.0) * jnp.maximum(qy2 - qy1, 0.0))
        qiou = qint / (a_iv + a_sv - qint + 1e-9)
        emit2 = active & (mS > neg_inf) & jnp.logical_not(qiou[0] > thr)
        cx1s = jnp.where(emit2, sx1, pinf_vec)
        cy1s = jnp.where(emit2, sy1, pinf_vec)
        cx2s = jnp.where(emit2, sx2, ninf_vec)
        cy2s = jnp.where(emit2, sy2, ninf_vec)
        a_sv2 = (cx2s - cx1s) * (cy2s - cy1s)
        wg2 = jnp.where(emit2, gS.astype(jnp.int32), -1)

        def fused(si, c2):
            nv1, ni1, nv2, ni2 = c2
            d = pl.ds(si * _L, _L)
            a = alive_v[d]
            x1 = ox1_v[d]
            y1 = oy1_v[d]
            x2 = ox2_v[d]
            y2 = oy2_v[d]
            aj = aj_v[d]
            xx1 = jnp.maximum(bx1, x1)
            yy1 = jnp.maximum(by1, y1)
            xx2 = jnp.minimum(bx2, x2)
            yy2 = jnp.minimum(by2, y2)
            inter = (jnp.maximum(xx2 - xx1, 0.0)
                     * jnp.maximum(yy2 - yy1, 0.0))
            iou = inter / (a_iv + aj - inter + 1e-9)
            ux1 = jnp.maximum(cx1s, x1)
            uy1 = jnp.maximum(cy1s, y1)
            ux2 = jnp.minimum(cx2s, x2)
            uy2 = jnp.minimum(cy2s, y2)
            uint = (jnp.maximum(ux2 - ux1, 0.0)
                    * jnp.maximum(uy2 - uy1, 0.0))
            uiou = uint / (a_sv2 + aj - uint + 1e-9)
            ii = lane + si * _L
            gidx = ii + base
            kill = ((iou > thr) | (uiou > thr)
                    | (gidx == wg1) | (gidx == wg2))
            a = jnp.where(kill, neg_inf, a)
            alive_v[d] = a
            b1 = a > nv1
            dv = jnp.where(b1, nv1, a)
            di = jnp.where(b1, ni1, ii)
            b2 = dv > nv2
            return (jnp.where(b1, a, nv1), jnp.where(b1, ii, ni1),
                    jnp.where(b2, dv, nv2), jnp.where(b2, di, ni2))

        nmv1, nmi1, nmv2, nmi2 = plsc.parallel_loop(
            0, _NSL, unroll=8,
            carry=(ninf_vec, zero_i, ninf_vec, zero_i))(fused)

        @pl.when(active & (g == 0) & (count < _MAX_DET))
        def _():
            o_v[count] = plsc.load_gather(rb_v, [grpv, kWv, lane])

        count1 = count + jnp.where(active, 1, 0)

        @pl.when(emit2 & (g == 0) & (count1 < _MAX_DET))
        def _():
            o_v[count1] = plsc.load_gather(rb_v, [grpv, kSv, lane])

        count2 = count1 + jnp.where(emit2, 1, 0)
        done2 = (count2 >= _MAX_DET) | jnp.logical_not(active)
        return (t + 1, go2, count2, done2, nmv1, nmi1, nmv2, nmi2)

    lax.while_loop(step_cond, step,
                   (jnp.int32(0), jnp.bool_(True), jnp.int32(0),
                    jnp.bool_(False), mv10, mi10, mv20, mi20))

    @pl.when(g == 0)
    def _():
        pltpu.sync_copy(o_v, out_hbm.at[img])


def kernel(boxes, scores, labels, img_size, nms_thresh, class_weights):
    B, N, _ = boxes.shape
    Np = _GS * _SHARD
    pad = Np - N

    thr = jax.nn.sigmoid(jnp.asarray(nms_thresh, jnp.float32))
    imgf = jnp.asarray(img_size, jnp.float32)
    cw = jnp.asarray(class_weights, jnp.float32)
    scal = jnp.zeros((16,), jnp.float32)
    scal = scal.at[0].set(imgf).at[1].set(thr)
    scal = scal.at[2].set(cw[0]).at[3].set(cw[1])

    x1 = jnp.pad(boxes[:, :, 0], ((0, 0), (0, pad)))
    y1 = jnp.pad(boxes[:, :, 1], ((0, 0), (0, pad)))
    x2 = jnp.pad(boxes[:, :, 2], ((0, 0), (0, pad)))
    y2 = jnp.pad(boxes[:, :, 3], ((0, 0), (0, pad)))
    scp = jnp.pad(scores, ((0, 0), (0, pad)), constant_values=-1.0)
    lbp = jnp.pad(labels.astype(jnp.int32), ((0, 0), (0, pad)))

    mesh = plsc.VectorSubcoreMesh(core_axis_name="c", subcore_axis_name="s",
                                  num_cores=2, num_subcores=16)
    run = pl.kernel(
        _sc_body,
        out_type=jax.ShapeDtypeStruct((B, _OUTW, _L), jnp.float32),
        mesh=mesh,
        compiler_params=pltpu.CompilerParams(needs_layout_passes=False),
        scratch_types=[
            pltpu.VMEM((_L,), jnp.float32),
            pltpu.VMEM((_SHARD,), jnp.float32),
            pltpu.VMEM((_SHARD,), jnp.float32),
            pltpu.VMEM((_SHARD,), jnp.float32),
            pltpu.VMEM((_SHARD,), jnp.float32),
            pltpu.VMEM((_SHARD,), jnp.float32),
            pltpu.VMEM((_SHARD,), jnp.float32),
            pltpu.VMEM((_SHARD,), jnp.float32),
            pltpu.VMEM((_SHARD,), jnp.float32),
            pltpu.VMEM((_SHARD,), jnp.float32),
            pltpu.VMEM((_SHARD,), jnp.float32),
            pltpu.VMEM((_SHARD,), jnp.float32),
            pltpu.VMEM((_SHARD,), jnp.int32),
            pltpu.VMEM((2, _L), jnp.float32),
            pltpu.VMEM((2, 2 * _GS, _L), jnp.float32),
            pltpu.VMEM((_OUTW, _L), jnp.float32),
            pltpu.VMEM_SHARED((3, 2, 2 * _GS, _L), jnp.float32),
        ],
    )
    out = run(scal, x1, y1, x2, y2, scp, lbp)

    det = out[:, :_MAX_DET, :]
    out_boxes = det[:, :, 8:12]
    out_scores = det[:, :, 6]
    out_labels = det[:, :, 7].astype(jnp.int32)
    return out_boxes, out_scores, out_labels


# speedup vs baseline: 1.7170x; 1.0203x over previous
import jax
import jax.numpy as jnp
from jax import lax
from jax.experimental import pallas as pl
from jax.experimental.pallas import tpu as pltpu
from jax.experimental.pallas import tpu_sc as plsc

_MAX_DET = 300
_OUTW = 304
_SCORE_THRESH = 0.3
_L = 16
_GS = 8
_SHARD = 640
_NSL = _SHARD // _L
_BIG = 1 << 30


def _sc_body(scal_hbm, x1_hbm, y1_hbm, x2_hbm, y2_hbm, sc_hbm, lb_hbm,
             out_hbm,
             scal_v, ox1_v, oy1_v, ox2_v, oy2_v,
             cx1_v, cy1_v, cx2_v, cy2_v, aj_v, alive_v, sc_v, lb_v,
             rec_v, rb_v, o_v, pub_sh):
    c = lax.axis_index("c")
    s = lax.axis_index("s")
    grp = s // _GS
    g = s % _GS
    img = c * 2 + grp
    base = g * _SHARD
    lane = lax.iota(jnp.int32, _L)
    neg_inf = jnp.float32(-jnp.inf)
    pos_inf = jnp.float32(jnp.inf)
    ninf_vec = jnp.full((_L,), neg_inf, jnp.float32)
    pinf_vec = jnp.full((_L,), pos_inf, jnp.float32)
    zero_i = jnp.zeros((_L,), jnp.int32)

    pltpu.sync_copy(scal_hbm, scal_v)
    sv = scal_v[...]
    imgf = sv[0]
    thr = sv[1]
    cw0 = sv[2]
    cw1 = sv[3]
    pltpu.sync_copy(x1_hbm.at[img, pl.ds(base, _SHARD)], cx1_v)
    pltpu.sync_copy(y1_hbm.at[img, pl.ds(base, _SHARD)], cy1_v)
    pltpu.sync_copy(x2_hbm.at[img, pl.ds(base, _SHARD)], cx2_v)
    pltpu.sync_copy(y2_hbm.at[img, pl.ds(base, _SHARD)], cy2_v)
    pltpu.sync_copy(sc_hbm.at[img, pl.ds(base, _SHARD)], sc_v)
    pltpu.sync_copy(lb_hbm.at[img, pl.ds(base, _SHARD)], lb_v)

    def prep(si, c2):
        mv1, mi1, mv2, mi2 = c2
        d = pl.ds(si * _L, _L)
        lb = lb_v[d]
        lbf = lb.astype(jnp.float32)
        off = lbf * (imgf + 1.0)
        cx1 = jnp.minimum(jnp.maximum(cx1_v[d], 0.0), imgf)
        cy1 = jnp.minimum(jnp.maximum(cy1_v[d], 0.0), imgf)
        cx2 = jnp.minimum(jnp.maximum(cx2_v[d], 0.0), imgf)
        cy2 = jnp.minimum(jnp.maximum(cy2_v[d], 0.0), imgf)
        cx1_v[d] = cx1
        cy1_v[d] = cy1
        cx2_v[d] = cx2
        cy2_v[d] = cy2
        x1 = cx1 + off
        y1 = cy1 + off
        x2 = cx2 + off
        y2 = cy2 + off
        ox1_v[d] = x1
        oy1_v[d] = y1
        ox2_v[d] = x2
        oy2_v[d] = y2
        aj_v[d] = (x2 - x1) * (y2 - y1)
        sc = sc_v[d]
        w = sc * jnp.where(lb == 0, cw0, cw1)
        a = jnp.where(sc > _SCORE_THRESH, w, neg_inf)
        alive_v[d] = a
        ii = lane + si * _L
        b1 = a > mv1
        dv = jnp.where(b1, mv1, a)
        di = jnp.where(b1, mi1, ii)
        b2 = dv > mv2
        return (jnp.where(b1, a, mv1), jnp.where(b1, ii, mi1),
                jnp.where(b2, dv, mv2), jnp.where(b2, di, mi2))

    mv10, mi10, mv20, mi20 = lax.fori_loop(
        0, _NSL, prep, (ninf_vec, zero_i, ninf_vec, zero_i))

    @pl.when(g == 0)
    def _():
        z = jnp.where(lane == 7, jnp.float32(-1.0), jnp.float32(0.0))

        def zinit(si, _):
            o_v[si] = z
            return 0

        lax.fori_loop(0, _OUTW, zinit, 0)

    rec_v[0] = ninf_vec
    rec_v[1] = ninf_vec
    pltpu.sync_copy(rec_v, pub_sh.at[1, grp, pl.ds(2 * g, 2)])
    pltpu.sync_copy(rec_v, pub_sh.at[2, grp, pl.ds(2 * g, 2)])
    plsc.subcore_barrier()

    def build_rec(m, win):
        winv = jnp.full((_L,), win, jnp.int32)
        rec = jnp.full((_L,), m, jnp.float32)
        rec = jnp.where(lane == 1, (base + win).astype(jnp.float32), rec)
        rec = jnp.where(lane == 2, plsc.load_gather(ox1_v, [winv]), rec)
        rec = jnp.where(lane == 3, plsc.load_gather(oy1_v, [winv]), rec)
        rec = jnp.where(lane == 4, plsc.load_gather(ox2_v, [winv]), rec)
        rec = jnp.where(lane == 5, plsc.load_gather(oy2_v, [winv]), rec)
        rec = jnp.where(lane == 6, plsc.load_gather(sc_v, [winv]), rec)
        rec = jnp.where(lane == 7,
                        plsc.load_gather(lb_v, [winv]).astype(jnp.float32), rec)
        rec = jnp.where(lane == 8, plsc.load_gather(cx1_v, [winv]), rec)
        rec = jnp.where(lane == 9, plsc.load_gather(cy1_v, [winv]), rec)
        rec = jnp.where(lane == 10, plsc.load_gather(cx2_v, [winv]), rec)
        rec = jnp.where(lane == 11, plsc.load_gather(cy2_v, [winv]), rec)
        return rec

    def step_cond(carry):
        t, go, count, done, mv1, mi1, mv2, mi2 = carry
        return go & (t < _MAX_DET + 2)

    def step(carry):
        t, go, count, done, mv1, mi1, mv2, mi2 = carry
        m1 = jnp.max(mv1)
        i1 = jnp.min(jnp.where(mv1 == m1, mi1, _BIG))
        mv1x = jnp.where((mv1 == m1) & (mi1 == i1), neg_inf, mv1)
        bb = (mv2 > mv1x) | ((mv2 == mv1x) & (mi2 < mi1))
        vc = jnp.where(bb, mv2, mv1x)
        ic = jnp.where(bb, mi2, mi1)
        m2 = jnp.max(vc)
        i2 = jnp.min(jnp.where(vc == m2, ic, _BIG))
        donef = jnp.where(done, jnp.float32(1.0), jnp.float32(0.0))
        rec_v[0] = jnp.where(lane == 12, donef, build_rec(m1, i1))
        rec_v[1] = jnp.where(lane == 12, donef, build_rec(m2, i2))

        p = t % 2 + 1
        pltpu.sync_copy(rec_v, pub_sh.at[p, grp, pl.ds(2 * g, 2)])
        plsc.subcore_barrier()
        pltpu.sync_copy(pub_sh.at[p], rb_v)

        grpv = jnp.full((_L,), grp, jnp.int32)
        ogrpv = jnp.full((_L,), 1 - grp, jnp.int32)
        val16 = plsc.load_gather(rb_v, [grpv, lane, zero_i])
        gid16 = plsc.load_gather(rb_v, [grpv, lane, zero_i + 1])
        other_done = plsc.load_gather(rb_v, [ogrpv, zero_i, zero_i + 12])[0] > 0.5
        go2 = jnp.logical_not(done & other_done)
        mW = jnp.max(val16)
        active = mW > neg_inf
        gW = jnp.min(jnp.where(val16 == mW, gid16, pos_inf))
        kW = jnp.min(jnp.where((val16 == mW) & (gid16 == gW), lane, _L))
        kWv = jnp.full((_L,), kW, jnp.int32)
        val16x = jnp.where(lane == kW, neg_inf, val16)
        mS = jnp.max(val16x)
        gS = jnp.min(jnp.where(val16x == mS, gid16, pos_inf))
        kS = jnp.min(jnp.where((val16x == mS) & (gid16 == gS), lane, _L))
        kSv = jnp.full((_L,), kS, jnp.int32)

        bx1 = jnp.where(active, plsc.load_gather(rb_v, [grpv, kWv, zero_i + 2]), pinf_vec)
        by1 = jnp.where(active, plsc.load_gather(rb_v, [grpv, kWv, zero_i + 3]), pinf_vec)
        bx2 = jnp.where(active, plsc.load_gather(rb_v, [grpv, kWv, zero_i + 4]), ninf_vec)
        by2 = jnp.where(active, plsc.load_gather(rb_v, [grpv, kWv, zero_i + 5]), ninf_vec)
        a_iv = (bx2 - bx1) * (by2 - by1)
        wg1 = jnp.where(active, gW.astype(jnp.int32), -1)

        sx1 = plsc.load_gather(rb_v, [grpv, kSv, zero_i + 2])
        sy1 = plsc.load_gather(rb_v, [grpv, kSv, zero_i + 3])
        sx2 = plsc.load_gather(rb_v, [grpv, kSv, zero_i + 4])
        sy2 = plsc.load_gather(rb_v, [grpv, kSv, zero_i + 5])
        a_sv = (sx2 - sx1) * (sy2 - sy1)
        qx1 = jnp.maximum(bx1, sx1)
        qy1 = jnp.maximum(by1, sy1)
        qx2 = jnp.minimum(bx2, sx2)
        qy2 = jnp.minimum(by2, sy2)
        qint = (jnp.maximum(qx2 - qx1, 0.0) * jnp.maximum(qy2 - qy1, 0.0))
        qiou = qint / (a_iv + a_sv - qint + 1e-9)
        emit2 = active & (mS > neg_inf) & jnp.logical_not(qiou[0] > thr)
        cx1s = jnp.where(emit2, sx1, pinf_vec)
        cy1s = jnp.where(emit2, sy1, pinf_vec)
        cx2s = jnp.where(emit2, sx2, ninf_vec)
        cy2s = jnp.where(emit2, sy2, ninf_vec)
        a_sv2 = (cx2s - cx1s) * (cy2s - cy1s)
        wg2 = jnp.where(emit2, gS.astype(jnp.int32), -1)

        w1l = wg1 - base
        mine1 = (w1l >= 0) & (w1l < _SHARD)
        plsc.store_scatter(
            alive_v, [jnp.full((_L,), jnp.clip(w1l, 0, _SHARD - 1), jnp.int32)],
            ninf_vec, mask=(lane == 0) & mine1)
        w2l = wg2 - base
        mine2 = (w2l >= 0) & (w2l < _SHARD)
        plsc.store_scatter(
            alive_v, [jnp.full((_L,), jnp.clip(w2l, 0, _SHARD - 1), jnp.int32)],
            ninf_vec, mask=(lane == 0) & mine2)

        def fused(si, c2):
            nv1, ni1, nv2, ni2 = c2
            d = pl.ds(si * _L, _L)
            a = alive_v[d]
            x1 = ox1_v[d]
            y1 = oy1_v[d]
            x2 = ox2_v[d]
            y2 = oy2_v[d]
            aj = aj_v[d]
            xx1 = jnp.maximum(bx1, x1)
            yy1 = jnp.maximum(by1, y1)
            xx2 = jnp.minimum(bx2, x2)
            yy2 = jnp.minimum(by2, y2)
            inter = (jnp.maximum(xx2 - xx1, 0.0)
                     * jnp.maximum(yy2 - yy1, 0.0))
            iou = inter / (a_iv + aj - inter + 1e-9)
            ux1 = jnp.maximum(cx1s, x1)
            uy1 = jnp.maximum(cy1s, y1)
            ux2 = jnp.minimum(cx2s, x2)
            uy2 = jnp.minimum(cy2s, y2)
            uint = (jnp.maximum(ux2 - ux1, 0.0)
                    * jnp.maximum(uy2 - uy1, 0.0))
            uiou = uint / (a_sv2 + aj - uint + 1e-9)
            ii = lane + si * _L
            kill = (iou > thr) | (uiou > thr)
            a = jnp.where(kill, neg_inf, a)
            alive_v[d] = a
            b1 = a > nv1
            dv = jnp.where(b1, nv1, a)
            di = jnp.where(b1, ni1, ii)
            b2 = dv > nv2
            return (jnp.where(b1, a, nv1), jnp.where(b1, ii, ni1),
                    jnp.where(b2, dv, nv2), jnp.where(b2, di, ni2))

        nmv1, nmi1, nmv2, nmi2 = plsc.parallel_loop(
            0, _NSL, unroll=8,
            carry=(ninf_vec, zero_i, ninf_vec, zero_i))(fused)

        @pl.when(active & (g == 0) & (count < _MAX_DET))
        def _():
            o_v[count] = plsc.load_gather(rb_v, [grpv, kWv, lane])

        count1 = count + jnp.where(active, 1, 0)

        @pl.when(emit2 & (g == 0) & (count1 < _MAX_DET))
        def _():
            o_v[count1] = plsc.load_gather(rb_v, [grpv, kSv, lane])

        count2 = count1 + jnp.where(emit2, 1, 0)
        done2 = (count2 >= _MAX_DET) | jnp.logical_not(active)
        return (t + 1, go2, count2, done2, nmv1, nmi1, nmv2, nmi2)

    lax.while_loop(step_cond, step,
                   (jnp.int32(0), jnp.bool_(True), jnp.int32(0),
                    jnp.bool_(False), mv10, mi10, mv20, mi20))

    @pl.when(g == 0)
    def _():
        pltpu.sync_copy(o_v, out_hbm.at[img])


def kernel(boxes, scores, labels, img_size, nms_thresh, class_weights):
    B, N, _ = boxes.shape
    Np = _GS * _SHARD
    pad = Np - N

    thr = jax.nn.sigmoid(jnp.asarray(nms_thresh, jnp.float32))
    imgf = jnp.asarray(img_size, jnp.float32)
    cw = jnp.asarray(class_weights, jnp.float32)
    scal = jnp.zeros((16,), jnp.float32)
    scal = scal.at[0].set(imgf).at[1].set(thr)
    scal = scal.at[2].set(cw[0]).at[3].set(cw[1])

    x1 = jnp.pad(boxes[:, :, 0], ((0, 0), (0, pad)))
    y1 = jnp.pad(boxes[:, :, 1], ((0, 0), (0, pad)))
    x2 = jnp.pad(boxes[:, :, 2], ((0, 0), (0, pad)))
    y2 = jnp.pad(boxes[:, :, 3], ((0, 0), (0, pad)))
    scp = jnp.pad(scores, ((0, 0), (0, pad)), constant_values=-1.0)
    lbp = jnp.pad(labels.astype(jnp.int32), ((0, 0), (0, pad)))

    mesh = plsc.VectorSubcoreMesh(core_axis_name="c", subcore_axis_name="s",
                                  num_cores=2, num_subcores=16)
    run = pl.kernel(
        _sc_body,
        out_type=jax.ShapeDtypeStruct((B, _OUTW, _L), jnp.float32),
        mesh=mesh,
        compiler_params=pltpu.CompilerParams(needs_layout_passes=False),
        scratch_types=[
            pltpu.VMEM((_L,), jnp.float32),
            pltpu.VMEM((_SHARD,), jnp.float32),
            pltpu.VMEM((_SHARD,), jnp.float32),
            pltpu.VMEM((_SHARD,), jnp.float32),
            pltpu.VMEM((_SHARD,), jnp.float32),
            pltpu.VMEM((_SHARD,), jnp.float32),
            pltpu.VMEM((_SHARD,), jnp.float32),
            pltpu.VMEM((_SHARD,), jnp.float32),
            pltpu.VMEM((_SHARD,), jnp.float32),
            pltpu.VMEM((_SHARD,), jnp.float32),
            pltpu.VMEM((_SHARD,), jnp.float32),
            pltpu.VMEM((_SHARD,), jnp.float32),
            pltpu.VMEM((_SHARD,), jnp.int32),
            pltpu.VMEM((2, _L), jnp.float32),
            pltpu.VMEM((2, 2 * _GS, _L), jnp.float32),
            pltpu.VMEM((_OUTW, _L), jnp.float32),
            pltpu.VMEM_SHARED((3, 2, 2 * _GS, _L), jnp.float32),
        ],
    )
    out = run(scal, x1, y1, x2, y2, scp, lbp)

    det = out[:, :_MAX_DET, :]
    out_boxes = det[:, :, 8:12]
    out_scores = det[:, :, 6]
    out_labels = det[:, :, 7].astype(jnp.int32)
    return out_boxes, out_scores, out_labels
